# Initial kernel scaffold; baseline (speedup 1.0000x reference)
import jax
import jax.numpy as jnp
from jax.experimental import pallas as pl


def kernel(atom_features, bond_features, adjacency_matrix, batch_indices, atom_table, bond_table, Wmsg, bmsg, Wih, Whh, bih, bhh, Wpool, bpool):
    raise NotImplementedError("write your pallas kernel here")


# speedup vs baseline: 1.1206x; 1.1206x over previous
import jax
import jax.numpy as jnp
from jax.experimental import pallas as pl
from jax.experimental.pallas import tpu as pltpu

N_ATOMS_ = 4096
N_BONDS_ = 8192
EMBED_ = 32
HIDDEN_ = 256
DEPTH_ = 10
N_GRAPHS_ = 64

BM_ = 512
NB_ = N_ATOMS_ // BM_

_F32 = jnp.float32
_BF16 = jnp.bfloat16


def _split(x):
    hi = x.astype(_BF16)
    lo = (x - hi.astype(_F32)).astype(_BF16)
    return hi, lo


def _dot(a, b):
    return jnp.dot(a, b, preferred_element_type=_F32)


def _x3(a_hi, a_lo, b_hi, b_lo):
    a = jnp.concatenate([a_hi, a_hi, a_lo], axis=1)
    b = jnp.concatenate([b_hi, b_lo, b_hi], axis=0)
    return _dot(a, b)


def _fused_kernel(adj_ref, af_ref, bf_ref, bi_ref, atab_ref, btab_ref,
                  wmsg_ref, bmsg_ref, wih_ref, whh_ref, bih_ref, bhh_ref,
                  wpool_ref, bpool_ref,
                  h_out_ref, bond_out_ref, graph_out_ref,
                  hbuf_ref, deg_ref):
    l = pl.program_id(0)
    m = pl.program_id(1)
    cur = jax.lax.rem(l, 2)
    nxt = 1 - cur
    rows = pl.ds(m * BM_, BM_)

    @pl.when(jnp.logical_and(l == 0, m == 0))
    def _prologue():
        af = af_ref[...]
        codes = jax.lax.broadcasted_iota(jnp.int32, (1, 256), 1)
        oh_a = (af == codes).astype(_BF16)
        atab = atab_ref[...]
        atab_p = jnp.concatenate(
            [atab, jnp.zeros((56, EMBED_), _F32)], axis=0)
        t_hi, t_lo = _split(atab_p)
        hbuf_ref[0] = _dot(oh_a, t_hi) + _dot(oh_a, t_lo)

    h_full = hbuf_ref[cur]
    h_hi, h_lo = _split(h_full)
    ones_col = jnp.ones((N_ATOMS_, 1), _BF16)
    x_nbr = jnp.concatenate([h_hi, h_lo, ones_col], axis=1)
    s = _dot(adj_ref[...], x_nbr)

    @pl.when(l == 0)
    def _store_deg():
        deg_ref[rows, :] = s[:, 2 * EMBED_:2 * EMBED_ + 1]

    deg = deg_ref[rows, :]
    has_nbr = deg > 0.0
    inv_deg = 1.0 / jnp.maximum(deg, 1.0)
    nbr_mean = (s[:, :EMBED_] + s[:, EMBED_:2 * EMBED_]) * inv_deg

    w1w2b_t = jnp.concatenate(
        [wmsg_ref[...][0], bmsg_ref[...][0].T], axis=1).T
    wih_t = wih_ref[...][0].T
    wa_hi, wa_lo = _split(w1w2b_t)
    wi_hi, wi_lo = _split(wih_t)
    ct = _x3(wa_hi, wa_lo, wi_hi, wi_lo)
    c1t = ct[:EMBED_, :]
    c2t = ct[EMBED_:2 * EMBED_, :]
    bih_l = bih_ref[...][0]
    d_l = bih_l + ct[2 * EMBED_:2 * EMBED_ + 1, :]
    whht = whh_ref[...][0].T

    hb = hbuf_ref[cur, rows, :]
    hb_hi, hb_lo = _split(hb)
    m_hi, m_lo = _split(nbr_mean)
    x = jnp.concatenate([hb_hi, hb_hi, hb_lo, m_hi, m_hi, m_lo], axis=1)
    c1_hi, c1_lo = _split(c1t)
    c2_hi, c2_lo = _split(c2t)
    wh_hi, wh_lo = _split(whht)
    zer = jnp.zeros((EMBED_, 96), _BF16)
    w_gi = jnp.concatenate([c1_hi, c1_lo, c1_hi, c2_hi, c2_lo, c2_hi], axis=0)
    w_gh = jnp.concatenate([wh_hi, wh_lo, wh_hi, zer, zer, zer], axis=0)
    g = _dot(x, jnp.concatenate([w_gi, w_gh], axis=1))

    gi = jnp.where(has_nbr, g[:, :96] + d_l, bih_l)
    gh = g[:, 96:192] + bhh_ref[...][0]

    r = jax.nn.sigmoid(gi[:, :EMBED_] + gh[:, :EMBED_])
    z = jax.nn.sigmoid(gi[:, EMBED_:2 * EMBED_] + gh[:, EMBED_:2 * EMBED_])
    n = jnp.tanh(gi[:, 2 * EMBED_:] + r * gh[:, 2 * EMBED_:])
    h_new = (1.0 - z) * n + z * hb

    hbuf_ref[nxt, rows, :] = h_new
    h_out_ref[...] = h_new

    @pl.when(jnp.logical_and(l == DEPTH_ - 1, m == NB_ - 1))
    def _epilogue():
        hf = hbuf_ref[DEPTH_ % 2]
        hf_hi, hf_lo = _split(hf)
        bi = bi_ref[...]
        gids = jax.lax.broadcasted_iota(jnp.int32, (N_GRAPHS_, 1), 0)
        oh_g = (bi == gids).astype(_BF16)
        p = _dot(oh_g, jnp.concatenate([hf_hi, hf_lo, ones_col], axis=1))
        counts = p[:, 2 * EMBED_:2 * EMBED_ + 1]
        sums = p[:, :EMBED_] + p[:, EMBED_:2 * EMBED_]
        means = jnp.where(counts > 0.0, sums / jnp.maximum(counts, 1.0), 0.0)
        wpt = wpool_ref[...].T
        mm_hi, mm_lo = _split(means)
        wp_hi, wp_lo = _split(wpt)
        graph_out_ref[...] = _x3(mm_hi, mm_lo, wp_hi, wp_lo) + bpool_ref[...]

        bfeat = bf_ref[...]
        bcodes = jax.lax.broadcasted_iota(jnp.int32, (1, 16), 1)
        oh_b = (bfeat == bcodes).astype(_BF16)
        btab = btab_ref[...]
        btab_p = jnp.concatenate(
            [btab, jnp.zeros((6, EMBED_), _F32)], axis=0)
        bt_hi, bt_lo = _split(btab_p)
        bond_out_ref[...] = _dot(oh_b, bt_hi) + _dot(oh_b, bt_lo)


@jax.jit
def kernel(atom_features, bond_features, adjacency_matrix, batch_indices,
           atom_table, bond_table, Wmsg, bmsg, Wih, Whh, bih, bhh, Wpool,
           bpool):
    adj16 = adjacency_matrix.astype(_BF16)
    af = atom_features.astype(jnp.int32).reshape(N_ATOMS_, 1)
    bf = bond_features.astype(jnp.int32).reshape(N_BONDS_, 1)
    bi = batch_indices.astype(jnp.int32).reshape(1, N_ATOMS_)
    bpool2 = bpool.reshape(1, HIDDEN_)
    bmsg3 = bmsg.reshape(DEPTH_, 1, HIDDEN_)
    bih3 = bih.reshape(DEPTH_, 1, 3 * EMBED_)
    bhh3 = bhh.reshape(DEPTH_, 1, 3 * EMBED_)

    def _full(arr):
        shape = arr.shape
        nd = len(shape)
        return pl.BlockSpec(shape, lambda l, m, _nd=nd: (0,) * _nd)

    out_shape = (
        jax.ShapeDtypeStruct((N_ATOMS_, EMBED_), _F32),
        jax.ShapeDtypeStruct((N_BONDS_, EMBED_), _F32),
        jax.ShapeDtypeStruct((N_GRAPHS_, HIDDEN_), _F32),
    )

    def _lw(arr):
        shape = (1,) + arr.shape[1:]
        nd = len(arr.shape)
        return pl.BlockSpec(shape,
                            lambda l, m, _nd=nd: (l,) + (0,) * (_nd - 1))

    in_specs = [
        pl.BlockSpec((BM_, N_ATOMS_), lambda l, m: (m, 0)),
        _full(af), _full(bf), _full(bi),
        _full(atom_table), _full(bond_table),
        _lw(Wmsg), _lw(bmsg3), _lw(Wih), _lw(Whh), _lw(bih3), _lw(bhh3),
        _full(Wpool), _full(bpool2),
    ]
    out_specs = (
        pl.BlockSpec((BM_, EMBED_), lambda l, m: (m, 0)),
        pl.BlockSpec((N_BONDS_, EMBED_), lambda l, m: (0, 0)),
        pl.BlockSpec((N_GRAPHS_, HIDDEN_), lambda l, m: (0, 0)),
    )

    h, bond_emb, graph_emb = pl.pallas_call(
        _fused_kernel,
        grid=(DEPTH_, NB_),
        in_specs=in_specs,
        out_specs=out_specs,
        out_shape=out_shape,
        scratch_shapes=[
            pltpu.VMEM((2, N_ATOMS_, EMBED_), _F32),
            pltpu.VMEM((N_ATOMS_, 1), _F32),
        ],
        compiler_params=pltpu.CompilerParams(
            dimension_semantics=("arbitrary", "arbitrary"),
        ),
    )(adj16, af, bf, bi, atom_table, bond_table, Wmsg, bmsg3, Wih, Whh, bih3,
      bhh3, Wpool, bpool2)
    return (h, bond_emb, graph_emb)


# speedup vs baseline: 1.2651x; 1.1290x over previous
import jax
import jax.numpy as jnp
from jax.experimental import pallas as pl
from jax.experimental.pallas import tpu as pltpu

N_ATOMS_ = 4096
N_BONDS_ = 8192
EMBED_ = 32
HIDDEN_ = 256
DEPTH_ = 10
N_GRAPHS_ = 64

BM_ = 512
NB_ = N_ATOMS_ // BM_

_F32 = jnp.float32
_BF16 = jnp.bfloat16


def _split(x):
    hi = x.astype(_BF16)
    lo = (x - hi.astype(_F32)).astype(_BF16)
    return hi, lo


def _dot(a, b):
    return jnp.dot(a, b, preferred_element_type=_F32)


def _x3(a_hi, a_lo, b_hi, b_lo):
    a = jnp.concatenate([a_hi, a_hi, a_lo], axis=1)
    b = jnp.concatenate([b_hi, b_lo, b_hi], axis=0)
    return _dot(a, b)


def _fused_kernel(adj_ref, af_ref, bf_ref, bi_ref, atab_ref, btab_ref,
                  wmsg_ref, bmsg_ref, wih_ref, whh_ref, bih_ref, bhh_ref,
                  wpool_ref, bpool_ref,
                  h_out_ref, bond_out_ref, graph_out_ref,
                  hbuf_ref, deg_ref, xnbr_ref, wg_ref, dl_ref):
    l = pl.program_id(0)
    m = pl.program_id(1)
    cur = jax.lax.rem(l, 2)
    nxt = 1 - cur
    rows = pl.ds(m * BM_, BM_)

    @pl.when(jnp.logical_and(l == 0, m == 0))
    def _prologue():
        af = af_ref[...]
        codes = jax.lax.broadcasted_iota(jnp.int32, (1, 256), 1)
        oh_a = (af == codes).astype(_BF16)
        atab = atab_ref[...]
        atab_p = jnp.concatenate(
            [atab, jnp.zeros((56, EMBED_), _F32)], axis=0)
        t_hi, t_lo = _split(atab_p)
        hbuf_ref[0] = _dot(oh_a, t_hi) + _dot(oh_a, t_lo)

    ones_col = jnp.ones((N_ATOMS_, 1), _BF16)

    @pl.when(m == 0)
    def _layer_prep():
        h_full = hbuf_ref[cur]
        h_hi, h_lo = _split(h_full)
        xnbr_ref[...] = jnp.concatenate([h_hi, h_lo, ones_col], axis=1)

        w1w2b_t = jnp.concatenate(
            [wmsg_ref[...][0], bmsg_ref[...][0].T], axis=1).T
        wih_t = wih_ref[...][0].T
        wa_hi, wa_lo = _split(w1w2b_t)
        wi_hi, wi_lo = _split(wih_t)
        ct = _x3(wa_hi, wa_lo, wi_hi, wi_lo)
        c1t = ct[:EMBED_, :]
        c2t = ct[EMBED_:2 * EMBED_, :]
        dl_ref[...] = bih_ref[...][0] + ct[2 * EMBED_:2 * EMBED_ + 1, :]
        whht = whh_ref[...][0].T

        c1_hi, c1_lo = _split(c1t)
        c2_hi, c2_lo = _split(c2t)
        wh_hi, wh_lo = _split(whht)
        zer = jnp.zeros((EMBED_, 96), _BF16)
        w_gi = jnp.concatenate(
            [c1_hi, c1_lo, c1_hi, c2_hi, c2_lo, c2_hi], axis=0)
        w_gh = jnp.concatenate(
            [wh_hi, wh_lo, wh_hi, zer, zer, zer], axis=0)
        wg_ref[...] = jnp.concatenate([w_gi, w_gh], axis=1)

    s = _dot(adj_ref[...], xnbr_ref[...])

    @pl.when(l == 0)
    def _store_deg():
        deg_ref[rows, :] = s[:, 2 * EMBED_:2 * EMBED_ + 1]

    deg = deg_ref[rows, :]
    has_nbr = deg > 0.0
    inv_deg = 1.0 / jnp.maximum(deg, 1.0)
    nbr_mean = (s[:, :EMBED_] + s[:, EMBED_:2 * EMBED_]) * inv_deg

    hb = hbuf_ref[cur, rows, :]
    hb_hi = xnbr_ref[rows, :EMBED_]
    hb_lo = xnbr_ref[rows, EMBED_:2 * EMBED_]
    m_hi, m_lo = _split(nbr_mean)
    x = jnp.concatenate([hb_hi, hb_hi, hb_lo, m_hi, m_hi, m_lo], axis=1)
    g = _dot(x, wg_ref[...])

    gi = jnp.where(has_nbr, g[:, :96] + dl_ref[...], bih_ref[...][0])
    gh = g[:, 96:192] + bhh_ref[...][0]

    r = jax.nn.sigmoid(gi[:, :EMBED_] + gh[:, :EMBED_])
    z = jax.nn.sigmoid(gi[:, EMBED_:2 * EMBED_] + gh[:, EMBED_:2 * EMBED_])
    n = jnp.tanh(gi[:, 2 * EMBED_:] + r * gh[:, 2 * EMBED_:])
    h_new = (1.0 - z) * n + z * hb

    hbuf_ref[nxt, rows, :] = h_new
    h_out_ref[...] = h_new

    @pl.when(jnp.logical_and(l == DEPTH_ - 1, m == NB_ - 1))
    def _epilogue():
        hf = hbuf_ref[DEPTH_ % 2]
        hf_hi, hf_lo = _split(hf)
        bi = bi_ref[...]
        gids = jax.lax.broadcasted_iota(jnp.int32, (N_GRAPHS_, 1), 0)
        oh_g = (bi == gids).astype(_BF16)
        p = _dot(oh_g, jnp.concatenate([hf_hi, hf_lo, ones_col], axis=1))
        counts = p[:, 2 * EMBED_:2 * EMBED_ + 1]
        sums = p[:, :EMBED_] + p[:, EMBED_:2 * EMBED_]
        means = jnp.where(counts > 0.0, sums / jnp.maximum(counts, 1.0), 0.0)
        wpt = wpool_ref[...].T
        mm_hi, mm_lo = _split(means)
        wp_hi, wp_lo = _split(wpt)
        graph_out_ref[...] = _x3(mm_hi, mm_lo, wp_hi, wp_lo) + bpool_ref[...]

        bfeat = bf_ref[...]
        bcodes = jax.lax.broadcasted_iota(jnp.int32, (1, 16), 1)
        oh_b = (bfeat == bcodes).astype(_BF16)
        btab = btab_ref[...]
        btab_p = jnp.concatenate(
            [btab, jnp.zeros((6, EMBED_), _F32)], axis=0)
        bt_hi, bt_lo = _split(btab_p)
        bond_out_ref[...] = _dot(oh_b, bt_hi) + _dot(oh_b, bt_lo)


@jax.jit
def kernel(atom_features, bond_features, adjacency_matrix, batch_indices,
           atom_table, bond_table, Wmsg, bmsg, Wih, Whh, bih, bhh, Wpool,
           bpool):
    adj16 = adjacency_matrix.astype(_BF16)
    af = atom_features.astype(jnp.int32).reshape(N_ATOMS_, 1)
    bf = bond_features.astype(jnp.int32).reshape(N_BONDS_, 1)
    bi = batch_indices.astype(jnp.int32).reshape(1, N_ATOMS_)
    bpool2 = bpool.reshape(1, HIDDEN_)
    bmsg3 = bmsg.reshape(DEPTH_, 1, HIDDEN_)
    bih3 = bih.reshape(DEPTH_, 1, 3 * EMBED_)
    bhh3 = bhh.reshape(DEPTH_, 1, 3 * EMBED_)

    def _full(arr):
        shape = arr.shape
        nd = len(shape)
        return pl.BlockSpec(shape, lambda l, m, _nd=nd: (0,) * _nd)

    out_shape = (
        jax.ShapeDtypeStruct((N_ATOMS_, EMBED_), _F32),
        jax.ShapeDtypeStruct((N_BONDS_, EMBED_), _F32),
        jax.ShapeDtypeStruct((N_GRAPHS_, HIDDEN_), _F32),
    )

    def _lw(arr):
        shape = (1,) + arr.shape[1:]
        nd = len(arr.shape)
        return pl.BlockSpec(shape,
                            lambda l, m, _nd=nd: (l,) + (0,) * (_nd - 1))

    in_specs = [
        pl.BlockSpec((BM_, N_ATOMS_), lambda l, m: (m, 0)),
        _full(af), _full(bf), _full(bi),
        _full(atom_table), _full(bond_table),
        _lw(Wmsg), _lw(bmsg3), _lw(Wih), _lw(Whh), _lw(bih3), _lw(bhh3),
        _full(Wpool), _full(bpool2),
    ]
    out_specs = (
        pl.BlockSpec((BM_, EMBED_), lambda l, m: (m, 0)),
        pl.BlockSpec((N_BONDS_, EMBED_), lambda l, m: (0, 0)),
        pl.BlockSpec((N_GRAPHS_, HIDDEN_), lambda l, m: (0, 0)),
    )

    h, bond_emb, graph_emb = pl.pallas_call(
        _fused_kernel,
        grid=(DEPTH_, NB_),
        in_specs=in_specs,
        out_specs=out_specs,
        out_shape=out_shape,
        scratch_shapes=[
            pltpu.VMEM((2, N_ATOMS_, EMBED_), _F32),
            pltpu.VMEM((N_ATOMS_, 1), _F32),
            pltpu.VMEM((N_ATOMS_, 2 * EMBED_ + 1), _BF16),
            pltpu.VMEM((6 * EMBED_, 6 * EMBED_), _BF16),
            pltpu.VMEM((1, 3 * EMBED_), _F32),
        ],
        compiler_params=pltpu.CompilerParams(
            dimension_semantics=("arbitrary", "arbitrary"),
        ),
    )(adj16, af, bf, bi, atom_table, bond_table, Wmsg, bmsg3, Wih, Whh, bih3,
      bhh3, Wpool, bpool2)
    return (h, bond_emb, graph_emb)


# speedup vs baseline: 1.4334x; 1.1330x over previous
import jax
import jax.numpy as jnp
from jax.experimental import pallas as pl
from jax.experimental.pallas import tpu as pltpu

N_ATOMS_ = 4096
N_BONDS_ = 8192
EMBED_ = 32
HIDDEN_ = 256
DEPTH_ = 10
N_GRAPHS_ = 64
NPART_ = 5

BM_ = 512
NB_ = N_ATOMS_ // BM_

_F32 = jnp.float32
_BF16 = jnp.bfloat16
_F8 = jnp.float8_e4m3fn


def _split(x):
    hi = x.astype(_BF16)
    lo = (x - hi.astype(_F32)).astype(_BF16)
    return hi, lo


def _dot(a, b):
    return jnp.dot(a, b, preferred_element_type=_F32)


def _x3(a_hi, a_lo, b_hi, b_lo):
    a = jnp.concatenate([a_hi, a_hi, a_lo], axis=1)
    b = jnp.concatenate([b_hi, b_lo, b_hi], axis=0)
    return _dot(a, b)


def _fused_kernel(adj_ref, af_ref, bf_ref, bi_ref, atab_ref, btab_ref,
                  wmsg_ref, bmsg_ref, wih_ref, whh_ref, bih_ref, bhh_ref,
                  wpool_ref, bpool_ref,
                  h_out_ref, bond_out_ref, graph_out_ref,
                  hbuf_ref, deg_ref, xnbr_ref, hsplit_ref, wg_ref, dl_ref):
    l = pl.program_id(0)
    m = pl.program_id(1)
    cur = jax.lax.rem(l, 2)
    nxt = 1 - cur
    rows = pl.ds(m * BM_, BM_)

    @pl.when(jnp.logical_and(l == 0, m == 0))
    def _prologue():
        af = af_ref[...]
        codes = jax.lax.broadcasted_iota(jnp.int32, (1, 256), 1)
        oh_a = (af == codes).astype(_BF16)
        atab = atab_ref[...]
        atab_p = jnp.concatenate(
            [atab, jnp.zeros((56, EMBED_), _F32)], axis=0)
        t_hi, t_lo = _split(atab_p)
        hbuf_ref[0] = _dot(oh_a, t_hi) + _dot(oh_a, t_lo)

    ones_col = jnp.ones((N_ATOMS_, 1), _BF16)

    @pl.when(m == 0)
    def _layer_prep():
        h_full = hbuf_ref[cur]
        h_hi, h_lo = _split(h_full)
        parts = []
        r = h_full
        for k in range(NPART_):
            pk = (r * (2.0 ** (4 * k))).astype(_F8)
            parts.append(pk)
            if k + 1 < NPART_:
                r = r - pk.astype(_F32) * (2.0 ** (-4 * k))
        parts.append(jnp.ones((N_ATOMS_, 1), _F8))
        xnbr_ref[...] = jnp.concatenate(parts, axis=1)
        hsplit_ref[...] = jnp.concatenate([h_hi, h_lo], axis=1)

        w1w2b_t = jnp.concatenate(
            [wmsg_ref[...][0], bmsg_ref[...][0].T], axis=1).T
        wih_t = wih_ref[...][0].T
        wa_hi, wa_lo = _split(w1w2b_t)
        wi_hi, wi_lo = _split(wih_t)
        ct = _x3(wa_hi, wa_lo, wi_hi, wi_lo)
        c1t = ct[:EMBED_, :]
        c2t = ct[EMBED_:2 * EMBED_, :]
        dl_ref[...] = bih_ref[...][0] + ct[2 * EMBED_:2 * EMBED_ + 1, :]
        whht = whh_ref[...][0].T

        c1_hi, c1_lo = _split(c1t)
        c2_hi, c2_lo = _split(c2t)
        wh_hi, wh_lo = _split(whht)
        zer = jnp.zeros((EMBED_, 96), _BF16)
        w_gi = jnp.concatenate(
            [c1_hi, c1_lo, c1_hi, c2_hi, c2_lo, c2_hi], axis=0)
        w_gh = jnp.concatenate(
            [wh_hi, wh_lo, wh_hi, zer, zer, zer], axis=0)
        wg_ref[...] = jnp.concatenate([w_gi, w_gh], axis=1)

    s = _dot(adj_ref[...], xnbr_ref[...])
    nbr_sum = s[:, :EMBED_]
    for k in range(1, NPART_):
        nbr_sum = nbr_sum + s[:, k * EMBED_:(k + 1) * EMBED_] * (2.0 ** (-4 * k))

    @pl.when(l == 0)
    def _store_deg():
        deg_ref[rows, :] = s[:, NPART_ * EMBED_:NPART_ * EMBED_ + 1]

    deg = deg_ref[rows, :]
    has_nbr = deg > 0.0
    inv_deg = 1.0 / jnp.maximum(deg, 1.0)
    nbr_mean = nbr_sum * inv_deg

    hb = hbuf_ref[cur, rows, :]
    hb_hi = hsplit_ref[rows, :EMBED_]
    hb_lo = hsplit_ref[rows, EMBED_:2 * EMBED_]
    m_hi, m_lo = _split(nbr_mean)
    x = jnp.concatenate([hb_hi, hb_hi, hb_lo, m_hi, m_hi, m_lo], axis=1)
    g = _dot(x, wg_ref[...])

    gi = jnp.where(has_nbr, g[:, :96] + dl_ref[...], bih_ref[...][0])
    gh = g[:, 96:192] + bhh_ref[...][0]

    r = jax.nn.sigmoid(gi[:, :EMBED_] + gh[:, :EMBED_])
    z = jax.nn.sigmoid(gi[:, EMBED_:2 * EMBED_] + gh[:, EMBED_:2 * EMBED_])
    n = jnp.tanh(gi[:, 2 * EMBED_:] + r * gh[:, 2 * EMBED_:])
    h_new = (1.0 - z) * n + z * hb

    hbuf_ref[nxt, rows, :] = h_new
    h_out_ref[...] = h_new

    @pl.when(jnp.logical_and(l == DEPTH_ - 1, m == NB_ - 1))
    def _epilogue():
        hf = hbuf_ref[DEPTH_ % 2]
        hf_hi, hf_lo = _split(hf)
        bi = bi_ref[...]
        gids = jax.lax.broadcasted_iota(jnp.int32, (N_GRAPHS_, 1), 0)
        oh_g = (bi == gids).astype(_BF16)
        p = _dot(oh_g, jnp.concatenate([hf_hi, hf_lo, ones_col], axis=1))
        counts = p[:, 2 * EMBED_:2 * EMBED_ + 1]
        sums = p[:, :EMBED_] + p[:, EMBED_:2 * EMBED_]
        means = jnp.where(counts > 0.0, sums / jnp.maximum(counts, 1.0), 0.0)
        wpt = wpool_ref[...].T
        mm_hi, mm_lo = _split(means)
        wp_hi, wp_lo = _split(wpt)
        graph_out_ref[...] = _x3(mm_hi, mm_lo, wp_hi, wp_lo) + bpool_ref[...]

        bfeat = bf_ref[...]
        bcodes = jax.lax.broadcasted_iota(jnp.int32, (1, 16), 1)
        oh_b = (bfeat == bcodes).astype(_BF16)
        btab = btab_ref[...]
        btab_p = jnp.concatenate(
            [btab, jnp.zeros((6, EMBED_), _F32)], axis=0)
        bt_hi, bt_lo = _split(btab_p)
        bond_out_ref[...] = _dot(oh_b, bt_hi) + _dot(oh_b, bt_lo)


@jax.jit
def kernel(atom_features, bond_features, adjacency_matrix, batch_indices,
           atom_table, bond_table, Wmsg, bmsg, Wih, Whh, bih, bhh, Wpool,
           bpool):
    adj16 = adjacency_matrix.astype(_F8)
    af = atom_features.astype(jnp.int32).reshape(N_ATOMS_, 1)
    bf = bond_features.astype(jnp.int32).reshape(N_BONDS_, 1)
    bi = batch_indices.astype(jnp.int32).reshape(1, N_ATOMS_)
    bpool2 = bpool.reshape(1, HIDDEN_)
    bmsg3 = bmsg.reshape(DEPTH_, 1, HIDDEN_)
    bih3 = bih.reshape(DEPTH_, 1, 3 * EMBED_)
    bhh3 = bhh.reshape(DEPTH_, 1, 3 * EMBED_)

    def _full(arr):
        shape = arr.shape
        nd = len(shape)
        return pl.BlockSpec(shape, lambda l, m, _nd=nd: (0,) * _nd)

    out_shape = (
        jax.ShapeDtypeStruct((N_ATOMS_, EMBED_), _F32),
        jax.ShapeDtypeStruct((N_BONDS_, EMBED_), _F32),
        jax.ShapeDtypeStruct((N_GRAPHS_, HIDDEN_), _F32),
    )

    def _lw(arr):
        shape = (1,) + arr.shape[1:]
        nd = len(arr.shape)
        return pl.BlockSpec(shape,
                            lambda l, m, _nd=nd: (l,) + (0,) * (_nd - 1))

    in_specs = [
        pl.BlockSpec((BM_, N_ATOMS_), lambda l, m: (m, 0)),
        _full(af), _full(bf), _full(bi),
        _full(atom_table), _full(bond_table),
        _lw(Wmsg), _lw(bmsg3), _lw(Wih), _lw(Whh), _lw(bih3), _lw(bhh3),
        _full(Wpool), _full(bpool2),
    ]
    out_specs = (
        pl.BlockSpec((BM_, EMBED_), lambda l, m: (m, 0)),
        pl.BlockSpec((N_BONDS_, EMBED_), lambda l, m: (0, 0)),
        pl.BlockSpec((N_GRAPHS_, HIDDEN_), lambda l, m: (0, 0)),
    )

    h, bond_emb, graph_emb = pl.pallas_call(
        _fused_kernel,
        grid=(DEPTH_, NB_),
        in_specs=in_specs,
        out_specs=out_specs,
        out_shape=out_shape,
        scratch_shapes=[
            pltpu.VMEM((2, N_ATOMS_, EMBED_), _F32),
            pltpu.VMEM((N_ATOMS_, 1), _F32),
            pltpu.VMEM((N_ATOMS_, NPART_ * EMBED_ + 1), _F8),
            pltpu.VMEM((N_ATOMS_, 2 * EMBED_), _BF16),
            pltpu.VMEM((6 * EMBED_, 6 * EMBED_), _BF16),
            pltpu.VMEM((1, 3 * EMBED_), _F32),
        ],
        compiler_params=pltpu.CompilerParams(
            dimension_semantics=("arbitrary", "arbitrary"),
        ),
    )(adj16, af, bf, bi, atom_table, bond_table, Wmsg, bmsg3, Wih, Whh, bih3,
      bhh3, Wpool, bpool2)
    return (h, bond_emb, graph_emb)


# speedup vs baseline: 1.4412x; 1.0054x over previous
import jax
import jax.numpy as jnp
from jax.experimental import pallas as pl
from jax.experimental.pallas import tpu as pltpu

N_ATOMS_ = 4096
N_BONDS_ = 8192
EMBED_ = 32
HIDDEN_ = 256
DEPTH_ = 10
N_GRAPHS_ = 64
NPART_ = 5
SCOL_ = NPART_ * EMBED_ + 1

BM_ = 512
NB_ = N_ATOMS_ // BM_

_F32 = jnp.float32
_BF16 = jnp.bfloat16
_F8 = jnp.float8_e4m3fn


def _split(x):
    hi = x.astype(_BF16)
    lo = (x - hi.astype(_F32)).astype(_BF16)
    return hi, lo


def _dot(a, b):
    return jnp.dot(a, b, preferred_element_type=_F32)


def _x3(a_hi, a_lo, b_hi, b_lo):
    a = jnp.concatenate([a_hi, a_hi, a_lo], axis=1)
    b = jnp.concatenate([b_hi, b_lo, b_hi], axis=0)
    return _dot(a, b)


def _fused_kernel(adj_ref, af_ref, bf_ref, bi_ref, atab_ref, btab_ref,
                  wmsg_ref, bmsg_ref, wih_ref, whh_ref, bih_ref, bhh_ref,
                  wpool_ref, bpool_ref,
                  h_out_ref, bond_out_ref, graph_out_ref,
                  hbuf_ref, deg_ref, xnbr_ref, hsplit_ref, wg_ref, dl_ref,
                  s_ref):
    l = pl.program_id(0)
    m = pl.program_id(1)
    cur = jax.lax.rem(l, 2)
    nxt = 1 - cur
    rows = pl.ds(m * BM_, BM_)
    first = jnp.logical_and(l == 0, m == 0)
    last = jnp.logical_and(l == DEPTH_ - 1, m == NB_ - 1)

    @pl.when(first)
    def _prologue():
        af = af_ref[...]
        codes = jax.lax.broadcasted_iota(jnp.int32, (1, 256), 1)
        oh_a = (af == codes).astype(_BF16)
        atab = atab_ref[...]
        atab_p = jnp.concatenate(
            [atab, jnp.zeros((56, EMBED_), _F32)], axis=0)
        t_hi, t_lo = _split(atab_p)
        hbuf_ref[0] = _dot(oh_a, t_hi) + _dot(oh_a, t_lo)

    def _gru_from_s(s_val, rows_sl, src_slot, dst_slot):
        nbr_sum = s_val[:, :EMBED_]
        for k in range(1, NPART_):
            nbr_sum = nbr_sum + (s_val[:, k * EMBED_:(k + 1) * EMBED_]
                                 * (2.0 ** (-4 * k)))
        deg = deg_ref[rows_sl, :]
        has_nbr = deg > 0.0
        inv_deg = 1.0 / jnp.maximum(deg, 1.0)
        nbr_mean = nbr_sum * inv_deg

        hb = hbuf_ref[src_slot, rows_sl, :]
        hb_hi = hsplit_ref[rows_sl, :EMBED_]
        hb_lo = hsplit_ref[rows_sl, EMBED_:2 * EMBED_]
        m_hi, m_lo = _split(nbr_mean)
        x = jnp.concatenate([hb_hi, hb_hi, hb_lo, m_hi, m_hi, m_lo], axis=1)
        g = _dot(x, wg_ref[...])

        d_l = dl_ref[0:1, :]
        bih_l = dl_ref[1:2, :]
        bhh_l = dl_ref[2:3, :]
        gi = jnp.where(has_nbr, g[:, :96] + d_l, bih_l)
        gh = g[:, 96:192] + bhh_l

        r = jax.nn.sigmoid(gi[:, :EMBED_] + gh[:, :EMBED_])
        z = jax.nn.sigmoid(gi[:, EMBED_:2 * EMBED_]
                           + gh[:, EMBED_:2 * EMBED_])
        n = jnp.tanh(gi[:, 2 * EMBED_:] + r * gh[:, 2 * EMBED_:])
        h_new = (1.0 - z) * n + z * hb
        hbuf_ref[dst_slot, rows_sl, :] = h_new

    @pl.when(jnp.logical_not(first))
    def _delayed():
        pm = jax.lax.rem(m + NB_ - 1, NB_)
        psrc = jnp.where(m > 0, cur, nxt)
        _gru_from_s(s_ref[...], pl.ds(pm * BM_, BM_), psrc, 1 - psrc)

    ones_col = jnp.ones((N_ATOMS_, 1), _BF16)

    @pl.when(m == 0)
    def _layer_prep():
        h_full = hbuf_ref[cur]
        h_hi, h_lo = _split(h_full)
        parts = []
        r = h_full
        for k in range(NPART_):
            pk = (r * (2.0 ** (4 * k))).astype(_F8)
            parts.append(pk)
            if k + 1 < NPART_:
                r = r - pk.astype(_F32) * (2.0 ** (-4 * k))
        parts.append(jnp.ones((N_ATOMS_, 1), _F8))
        xnbr_ref[...] = jnp.concatenate(parts, axis=1)
        hsplit_ref[...] = jnp.concatenate([h_hi, h_lo], axis=1)

        w1w2b_t = jnp.concatenate(
            [wmsg_ref[...][0], bmsg_ref[...][0].T], axis=1).T
        wih_t = wih_ref[...][0].T
        wa_hi, wa_lo = _split(w1w2b_t)
        wi_hi, wi_lo = _split(wih_t)
        ct = _x3(wa_hi, wa_lo, wi_hi, wi_lo)
        c1t = ct[:EMBED_, :]
        c2t = ct[EMBED_:2 * EMBED_, :]
        bih_l = bih_ref[...][0]
        dl_ref[...] = jnp.concatenate(
            [bih_l + ct[2 * EMBED_:2 * EMBED_ + 1, :], bih_l,
             bhh_ref[...][0]], axis=0)
        whht = whh_ref[...][0].T

        c1_hi, c1_lo = _split(c1t)
        c2_hi, c2_lo = _split(c2t)
        wh_hi, wh_lo = _split(whht)
        zer = jnp.zeros((EMBED_, 96), _BF16)
        w_gi = jnp.concatenate(
            [c1_hi, c1_lo, c1_hi, c2_hi, c2_lo, c2_hi], axis=0)
        w_gh = jnp.concatenate(
            [wh_hi, wh_lo, wh_hi, zer, zer, zer], axis=0)
        wg_ref[...] = jnp.concatenate([w_gi, w_gh], axis=1)

    s = _dot(adj_ref[...], xnbr_ref[...])
    s_ref[...] = s

    @pl.when(l == 0)
    def _store_deg():
        deg_ref[rows, :] = s[:, NPART_ * EMBED_:NPART_ * EMBED_ + 1]

    @pl.when(last)
    def _epilogue():
        _gru_from_s(s, rows, cur, nxt)
        hf = hbuf_ref[DEPTH_ % 2]
        h_out_ref[...] = hf
        hf_hi, hf_lo = _split(hf)
        bi = bi_ref[...]
        gids = jax.lax.broadcasted_iota(jnp.int32, (N_GRAPHS_, 1), 0)
        oh_g = (bi == gids).astype(_BF16)
        p = _dot(oh_g, jnp.concatenate([hf_hi, hf_lo, ones_col], axis=1))
        counts = p[:, 2 * EMBED_:2 * EMBED_ + 1]
        sums = p[:, :EMBED_] + p[:, EMBED_:2 * EMBED_]
        means = jnp.where(counts > 0.0, sums / jnp.maximum(counts, 1.0), 0.0)
        wpt = wpool_ref[...].T
        mm_hi, mm_lo = _split(means)
        wp_hi, wp_lo = _split(wpt)
        graph_out_ref[...] = _x3(mm_hi, mm_lo, wp_hi, wp_lo) + bpool_ref[...]

        bfeat = bf_ref[...]
        bcodes = jax.lax.broadcasted_iota(jnp.int32, (1, 16), 1)
        oh_b = (bfeat == bcodes).astype(_BF16)
        btab = btab_ref[...]
        btab_p = jnp.concatenate(
            [btab, jnp.zeros((6, EMBED_), _F32)], axis=0)
        bt_hi, bt_lo = _split(btab_p)
        bond_out_ref[...] = _dot(oh_b, bt_hi) + _dot(oh_b, bt_lo)


@jax.jit
def kernel(atom_features, bond_features, adjacency_matrix, batch_indices,
           atom_table, bond_table, Wmsg, bmsg, Wih, Whh, bih, bhh, Wpool,
           bpool):
    adj16 = adjacency_matrix.astype(_F8)
    af = atom_features.astype(jnp.int32).reshape(N_ATOMS_, 1)
    bf = bond_features.astype(jnp.int32).reshape(N_BONDS_, 1)
    bi = batch_indices.astype(jnp.int32).reshape(1, N_ATOMS_)
    bpool2 = bpool.reshape(1, HIDDEN_)
    bmsg3 = bmsg.reshape(DEPTH_, 1, HIDDEN_)
    bih3 = bih.reshape(DEPTH_, 1, 3 * EMBED_)
    bhh3 = bhh.reshape(DEPTH_, 1, 3 * EMBED_)

    def _full(arr):
        shape = arr.shape
        nd = len(shape)
        return pl.BlockSpec(shape, lambda l, m, _nd=nd: (0,) * _nd)

    out_shape = (
        jax.ShapeDtypeStruct((N_ATOMS_, EMBED_), _F32),
        jax.ShapeDtypeStruct((N_BONDS_, EMBED_), _F32),
        jax.ShapeDtypeStruct((N_GRAPHS_, HIDDEN_), _F32),
    )

    def _lw(arr):
        shape = (1,) + arr.shape[1:]
        nd = len(arr.shape)
        return pl.BlockSpec(shape,
                            lambda l, m, _nd=nd: (l,) + (0,) * (_nd - 1))

    in_specs = [
        pl.BlockSpec((BM_, N_ATOMS_), lambda l, m: (m, 0)),
        _full(af), _full(bf), _full(bi),
        _full(atom_table), _full(bond_table),
        _lw(Wmsg), _lw(bmsg3), _lw(Wih), _lw(Whh), _lw(bih3), _lw(bhh3),
        _full(Wpool), _full(bpool2),
    ]
    out_specs = (
        pl.BlockSpec((N_ATOMS_, EMBED_), lambda l, m: (0, 0)),
        pl.BlockSpec((N_BONDS_, EMBED_), lambda l, m: (0, 0)),
        pl.BlockSpec((N_GRAPHS_, HIDDEN_), lambda l, m: (0, 0)),
    )

    h, bond_emb, graph_emb = pl.pallas_call(
        _fused_kernel,
        grid=(DEPTH_, NB_),
        in_specs=in_specs,
        out_specs=out_specs,
        out_shape=out_shape,
        scratch_shapes=[
            pltpu.VMEM((2, N_ATOMS_, EMBED_), _F32),
            pltpu.VMEM((N_ATOMS_, 1), _F32),
            pltpu.VMEM((N_ATOMS_, SCOL_), _F8),
            pltpu.VMEM((N_ATOMS_, 2 * EMBED_), _BF16),
            pltpu.VMEM((6 * EMBED_, 6 * EMBED_), _BF16),
            pltpu.VMEM((3, 3 * EMBED_), _F32),
            pltpu.VMEM((BM_, SCOL_), _F32),
        ],
        compiler_params=pltpu.CompilerParams(
            dimension_semantics=("arbitrary", "arbitrary"),
        ),
    )(adj16, af, bf, bi, atom_table, bond_table, Wmsg, bmsg3, Wih, Whh, bih3,
      bhh3, Wpool, bpool2)
    return (h, bond_emb, graph_emb)


# speedup vs baseline: 1.5432x; 1.0708x over previous
import jax
import jax.numpy as jnp
from jax.experimental import pallas as pl
from jax.experimental.pallas import tpu as pltpu

N_ATOMS_ = 4096
N_BONDS_ = 8192
EMBED_ = 32
HIDDEN_ = 256
DEPTH_ = 10
N_GRAPHS_ = 64
NPART_ = 5
SCOL_ = NPART_ * EMBED_ + 1

BM_ = 1024
NB_ = N_ATOMS_ // BM_

_F32 = jnp.float32
_BF16 = jnp.bfloat16
_F8 = jnp.float8_e4m3fn


def _split(x):
    hi = x.astype(_BF16)
    lo = (x - hi.astype(_F32)).astype(_BF16)
    return hi, lo


def _dot(a, b):
    return jnp.dot(a, b, preferred_element_type=_F32)


def _x3(a_hi, a_lo, b_hi, b_lo):
    a = jnp.concatenate([a_hi, a_hi, a_lo], axis=1)
    b = jnp.concatenate([b_hi, b_lo, b_hi], axis=0)
    return _dot(a, b)


def _fused_kernel(adj_ref, af_ref, bf_ref, bi_ref, atab_ref, btab_ref,
                  wmsg_ref, bmsg_ref, wih_ref, whh_ref, bih_ref, bhh_ref,
                  wpool_ref, bpool_ref,
                  h_out_ref, bond_out_ref, graph_out_ref,
                  hbuf_ref, deg_ref, xnbr_ref, hsplit_ref, wg_ref, dl_ref,
                  s_ref):
    l = pl.program_id(0)
    m = pl.program_id(1)
    cur = jax.lax.rem(l, 2)
    nxt = 1 - cur
    rows = pl.ds(m * BM_, BM_)
    first = jnp.logical_and(l == 0, m == 0)
    last = jnp.logical_and(l == DEPTH_ - 1, m == NB_ - 1)

    @pl.when(first)
    def _prologue():
        af = af_ref[...]
        codes = jax.lax.broadcasted_iota(jnp.int32, (1, 256), 1)
        oh_a = (af == codes).astype(_BF16)
        atab = atab_ref[...]
        atab_p = jnp.concatenate(
            [atab, jnp.zeros((56, EMBED_), _F32)], axis=0)
        t_hi, t_lo = _split(atab_p)
        hbuf_ref[0] = _dot(oh_a, t_hi) + _dot(oh_a, t_lo)

    def _gru_from_s(s_val, rows_sl, src_slot, dst_slot):
        nbr_sum = s_val[:, :EMBED_]
        for k in range(1, NPART_):
            nbr_sum = nbr_sum + (s_val[:, k * EMBED_:(k + 1) * EMBED_]
                                 * (2.0 ** (-4 * k)))
        deg = deg_ref[rows_sl, :]
        has_nbr = deg > 0.0
        inv_deg = 1.0 / jnp.maximum(deg, 1.0)
        nbr_mean = nbr_sum * inv_deg

        hb = hbuf_ref[src_slot, rows_sl, :]
        hb_hi = hsplit_ref[rows_sl, :EMBED_]
        hb_lo = hsplit_ref[rows_sl, EMBED_:2 * EMBED_]
        m_hi, m_lo = _split(nbr_mean)
        x = jnp.concatenate([hb_hi, hb_hi, hb_lo, m_hi, m_hi, m_lo], axis=1)
        g = _dot(x, wg_ref[...])

        d_l = dl_ref[0:1, :]
        bih_l = dl_ref[1:2, :]
        bhh_l = dl_ref[2:3, :]
        gi = jnp.where(has_nbr, g[:, :96] + d_l, bih_l)
        gh = g[:, 96:192] + bhh_l

        r = jax.nn.sigmoid(gi[:, :EMBED_] + gh[:, :EMBED_])
        z = jax.nn.sigmoid(gi[:, EMBED_:2 * EMBED_]
                           + gh[:, EMBED_:2 * EMBED_])
        n = jnp.tanh(gi[:, 2 * EMBED_:] + r * gh[:, 2 * EMBED_:])
        h_new = (1.0 - z) * n + z * hb
        hbuf_ref[dst_slot, rows_sl, :] = h_new

    @pl.when(jnp.logical_not(first))
    def _delayed():
        pm = jax.lax.rem(m + NB_ - 1, NB_)
        psrc = jnp.where(m > 0, cur, nxt)
        _gru_from_s(s_ref[...], pl.ds(pm * BM_, BM_), psrc, 1 - psrc)

    ones_col = jnp.ones((N_ATOMS_, 1), _BF16)

    @pl.when(m == 0)
    def _layer_prep():
        h_full = hbuf_ref[cur]
        h_hi, h_lo = _split(h_full)
        parts = []
        r = h_full
        for k in range(NPART_):
            pk = (r * (2.0 ** (4 * k))).astype(_F8)
            parts.append(pk)
            if k + 1 < NPART_:
                r = r - pk.astype(_F32) * (2.0 ** (-4 * k))
        parts.append(jnp.ones((N_ATOMS_, 1), _F8))
        xnbr_ref[...] = jnp.concatenate(parts, axis=1)
        hsplit_ref[...] = jnp.concatenate([h_hi, h_lo], axis=1)

        w1w2b_t = jnp.concatenate(
            [wmsg_ref[...][0], bmsg_ref[...][0].T], axis=1).T
        wih_t = wih_ref[...][0].T
        wa_hi, wa_lo = _split(w1w2b_t)
        wi_hi, wi_lo = _split(wih_t)
        ct = _x3(wa_hi, wa_lo, wi_hi, wi_lo)
        c1t = ct[:EMBED_, :]
        c2t = ct[EMBED_:2 * EMBED_, :]
        bih_l = bih_ref[...][0]
        dl_ref[...] = jnp.concatenate(
            [bih_l + ct[2 * EMBED_:2 * EMBED_ + 1, :], bih_l,
             bhh_ref[...][0]], axis=0)
        whht = whh_ref[...][0].T

        c1_hi, c1_lo = _split(c1t)
        c2_hi, c2_lo = _split(c2t)
        wh_hi, wh_lo = _split(whht)
        zer = jnp.zeros((EMBED_, 96), _BF16)
        w_gi = jnp.concatenate(
            [c1_hi, c1_lo, c1_hi, c2_hi, c2_lo, c2_hi], axis=0)
        w_gh = jnp.concatenate(
            [wh_hi, wh_lo, wh_hi, zer, zer, zer], axis=0)
        wg_ref[...] = jnp.concatenate([w_gi, w_gh], axis=1)

    s = _dot(adj_ref[...], xnbr_ref[...])
    s_ref[...] = s

    @pl.when(l == 0)
    def _store_deg():
        deg_ref[rows, :] = s[:, NPART_ * EMBED_:NPART_ * EMBED_ + 1]

    @pl.when(last)
    def _epilogue():
        _gru_from_s(s, rows, cur, nxt)
        hf = hbuf_ref[DEPTH_ % 2]
        h_out_ref[...] = hf
        hf_hi, hf_lo = _split(hf)
        bi = bi_ref[...]
        gids = jax.lax.broadcasted_iota(jnp.int32, (N_GRAPHS_, 1), 0)
        oh_g = (bi == gids).astype(_BF16)
        p = _dot(oh_g, jnp.concatenate([hf_hi, hf_lo, ones_col], axis=1))
        counts = p[:, 2 * EMBED_:2 * EMBED_ + 1]
        sums = p[:, :EMBED_] + p[:, EMBED_:2 * EMBED_]
        means = jnp.where(counts > 0.0, sums / jnp.maximum(counts, 1.0), 0.0)
        wpt = wpool_ref[...].T
        mm_hi, mm_lo = _split(means)
        wp_hi, wp_lo = _split(wpt)
        graph_out_ref[...] = _x3(mm_hi, mm_lo, wp_hi, wp_lo) + bpool_ref[...]

        bfeat = bf_ref[...]
        bcodes = jax.lax.broadcasted_iota(jnp.int32, (1, 16), 1)
        oh_b = (bfeat == bcodes).astype(_BF16)
        btab = btab_ref[...]
        btab_p = jnp.concatenate(
            [btab, jnp.zeros((6, EMBED_), _F32)], axis=0)
        bt_hi, bt_lo = _split(btab_p)
        bond_out_ref[...] = _dot(oh_b, bt_hi) + _dot(oh_b, bt_lo)


@jax.jit
def kernel(atom_features, bond_features, adjacency_matrix, batch_indices,
           atom_table, bond_table, Wmsg, bmsg, Wih, Whh, bih, bhh, Wpool,
           bpool):
    adj16 = adjacency_matrix.astype(_F8)
    af = atom_features.astype(jnp.int32).reshape(N_ATOMS_, 1)
    bf = bond_features.astype(jnp.int32).reshape(N_BONDS_, 1)
    bi = batch_indices.astype(jnp.int32).reshape(1, N_ATOMS_)
    bpool2 = bpool.reshape(1, HIDDEN_)
    bmsg3 = bmsg.reshape(DEPTH_, 1, HIDDEN_)
    bih3 = bih.reshape(DEPTH_, 1, 3 * EMBED_)
    bhh3 = bhh.reshape(DEPTH_, 1, 3 * EMBED_)

    def _full(arr):
        shape = arr.shape
        nd = len(shape)
        return pl.BlockSpec(shape, lambda l, m, _nd=nd: (0,) * _nd)

    out_shape = (
        jax.ShapeDtypeStruct((N_ATOMS_, EMBED_), _F32),
        jax.ShapeDtypeStruct((N_BONDS_, EMBED_), _F32),
        jax.ShapeDtypeStruct((N_GRAPHS_, HIDDEN_), _F32),
    )

    def _lw(arr):
        shape = (1,) + arr.shape[1:]
        nd = len(arr.shape)
        return pl.BlockSpec(shape,
                            lambda l, m, _nd=nd: (l,) + (0,) * (_nd - 1))

    in_specs = [
        pl.BlockSpec((BM_, N_ATOMS_), lambda l, m: (m, 0)),
        _full(af), _full(bf), _full(bi),
        _full(atom_table), _full(bond_table),
        _lw(Wmsg), _lw(bmsg3), _lw(Wih), _lw(Whh), _lw(bih3), _lw(bhh3),
        _full(Wpool), _full(bpool2),
    ]
    out_specs = (
        pl.BlockSpec((N_ATOMS_, EMBED_), lambda l, m: (0, 0)),
        pl.BlockSpec((N_BONDS_, EMBED_), lambda l, m: (0, 0)),
        pl.BlockSpec((N_GRAPHS_, HIDDEN_), lambda l, m: (0, 0)),
    )

    h, bond_emb, graph_emb = pl.pallas_call(
        _fused_kernel,
        grid=(DEPTH_, NB_),
        in_specs=in_specs,
        out_specs=out_specs,
        out_shape=out_shape,
        scratch_shapes=[
            pltpu.VMEM((2, N_ATOMS_, EMBED_), _F32),
            pltpu.VMEM((N_ATOMS_, 1), _F32),
            pltpu.VMEM((N_ATOMS_, SCOL_), _F8),
            pltpu.VMEM((N_ATOMS_, 2 * EMBED_), _BF16),
            pltpu.VMEM((6 * EMBED_, 6 * EMBED_), _BF16),
            pltpu.VMEM((3, 3 * EMBED_), _F32),
            pltpu.VMEM((BM_, SCOL_), _F32),
        ],
        compiler_params=pltpu.CompilerParams(
            dimension_semantics=("arbitrary", "arbitrary"),
        ),
    )(adj16, af, bf, bi, atom_table, bond_table, Wmsg, bmsg3, Wih, Whh, bih3,
      bhh3, Wpool, bpool2)
    return (h, bond_emb, graph_emb)
